# Initial kernel scaffold; baseline (speedup 1.0000x reference)
import jax
import jax.numpy as jnp
from jax.experimental import pallas as pl


def kernel(x):
    raise NotImplementedError("write your pallas kernel here")


# speedup vs baseline: 2.6262x; 2.6262x over previous
import functools

import jax
import jax.numpy as jnp
from jax import lax
from jax.experimental import pallas as pl
from jax.experimental.pallas import tpu as pltpu
from jax.experimental.pallas import tpu_sc as plsc

NC, NS, L = 2, 16, 16
ROWS, N = 64, 100000
ROWS_PER_SC = ROWS // NC
CHUNK = 6256
NPAD = NS * CHUNK
TAIL = NPAD - N
REAL_LAST = CHUNK - TAIL
NV = CHUNK // L
NBINS = 256
INT_MIN = jnp.int32(-2**31)

_mesh = None


def _get_mesh():
    global _mesh
    if _mesh is None:
        _mesh = plsc.VectorSubcoreMesh(
            core_axis_name="c", subcore_axis_name="s",
            num_cores=NC, num_subcores=NS)
    return _mesh


def _body(x_hbm, out_hbm, keys_sp, perm_a, perm_b, totals_sp,
          fbuf, kbuf, pbuf, posbuf, hist, offs, ttbuf):
    c = lax.axis_index("c")
    t = lax.axis_index("s")
    lane = lax.iota(jnp.int32, L)
    zeros16 = jnp.zeros((L,), jnp.int32)
    my_lo = t * CHUNK

    def digits(k, shift):
        return lax.shift_right_logical(k, shift) & 255

    def radix_pass(shift, perm_in, perm_out, first):
        if not first:
            pltpu.sync_copy(perm_in.at[pl.ds(my_lo, CHUNK)], pbuf)
            pltpu.sync_copy(keys_sp.at[pbuf], kbuf)
        for j in range(NBINS // L):
            hist[pl.ds(L * j, L)] = zeros16

        def hist_body(i, _):
            d = digits(kbuf[pl.ds(L * i, L)], shift)
            cnt, last = plsc.scan_count(d)
            plsc.addupdate_scatter(hist, [d], cnt, mask=last)
            return 0
        lax.fori_loop(0, NV, hist_body, 0)

        pltpu.sync_copy(hist, totals_sp.at[t])
        plsc.subcore_barrier()
        pltpu.sync_copy(totals_sp, ttbuf)

        carry = jnp.int32(0)
        for j in range(NBINS // L):
            sl = pl.ds(L * j, L)

            def tot_body(tp, acc):
                return acc + ttbuf[tp, sl]
            tot = lax.fori_loop(0, NS, tot_body, zeros16)
            below = lax.fori_loop(0, t, tot_body, zeros16)
            excl = plsc.cumsum(tot) - tot
            offs[sl] = excl + below + carry
            carry = carry + jnp.sum(tot)

        def pos_body(i, _):
            d = digits(kbuf[pl.ds(L * i, L)], shift)
            cnt, last = plsc.scan_count(d)
            off = plsc.load_gather(offs, [d])
            posbuf[pl.ds(L * i, L)] = off + cnt - 1
            plsc.addupdate_scatter(offs, [d], cnt, mask=last)
            return 0
        lax.fori_loop(0, NV, pos_body, 0)

        pltpu.sync_copy(pbuf, perm_out.at[posbuf])
        plsc.subcore_barrier()

    def row_body(r, _):
        row = c * ROWS_PER_SC + r

        @pl.when(t < NS - 1)
        def _():
            pltpu.sync_copy(x_hbm.at[row, pl.ds(my_lo, CHUNK)], fbuf)

        @pl.when(t == NS - 1)
        def _():
            pltpu.sync_copy(x_hbm.at[row, pl.ds((NS - 1) * CHUNK, REAL_LAST)],
                            fbuf.at[pl.ds(0, REAL_LAST)])

        def xform_body(i, _):
            b = plsc.bitcast(fbuf[pl.ds(L * i, L)], jnp.int32)
            key = b ^ (lax.shift_right_arithmetic(b, 31) | INT_MIN)
            kbuf[pl.ds(L * i, L)] = key
            pbuf[pl.ds(L * i, L)] = my_lo + L * i + lane
            return 0
        lax.fori_loop(0, NV, xform_body, 0)

        @pl.when(t == NS - 1)
        def _():
            for j in range(TAIL // L):
                kbuf[pl.ds(REAL_LAST + L * j, L)] = zeros16 - 1

        pltpu.sync_copy(kbuf, keys_sp.at[pl.ds(my_lo, CHUNK)])

        radix_pass(0, perm_a, perm_b, first=True)
        radix_pass(8, perm_b, perm_a, first=False)
        radix_pass(16, perm_a, perm_b, first=False)
        radix_pass(24, perm_b, perm_a, first=False)

        @pl.when(t < NS - 1)
        def _():
            pltpu.sync_copy(perm_a.at[pl.ds(my_lo, CHUNK)],
                            out_hbm.at[row, pl.ds(my_lo, CHUNK)])

        @pl.when(t == NS - 1)
        def _():
            pltpu.sync_copy(perm_a.at[pl.ds((NS - 1) * CHUNK, REAL_LAST)],
                            out_hbm.at[row, pl.ds((NS - 1) * CHUNK, REAL_LAST)])
        return 0

    lax.fori_loop(0, ROWS_PER_SC, row_body, 0)


@jax.jit
def kernel(x):
    run = functools.partial(
        pl.kernel,
        out_type=jax.ShapeDtypeStruct((ROWS, N), jnp.int32),
        mesh=_get_mesh(),
        scratch_types=[
            pltpu.VMEM_SHARED((NPAD,), jnp.int32),
            pltpu.VMEM_SHARED((NPAD,), jnp.int32),
            pltpu.VMEM_SHARED((NPAD,), jnp.int32),
            pltpu.VMEM_SHARED((NS, NBINS), jnp.int32),
            pltpu.VMEM((CHUNK,), jnp.float32),
            pltpu.VMEM((CHUNK,), jnp.int32),
            pltpu.VMEM((CHUNK,), jnp.int32),
            pltpu.VMEM((CHUNK,), jnp.int32),
            pltpu.VMEM((NBINS,), jnp.int32),
            pltpu.VMEM((NBINS,), jnp.int32),
            pltpu.VMEM((NS, NBINS), jnp.int32),
        ],
        compiler_params=pltpu.CompilerParams(
            needs_layout_passes=False, use_tc_tiling_on_sc=False),
    )(_body)
    return run(x)


# speedup vs baseline: 3.7742x; 1.4371x over previous
import functools

import jax
import jax.numpy as jnp
from jax import lax
from jax.experimental import pallas as pl
from jax.experimental.pallas import tpu as pltpu
from jax.experimental.pallas import tpu_sc as plsc

NC, NS, L = 2, 16, 16
ROWS, N = 64, 100000
ROWS_PER_SC = ROWS // NC
CHUNK = 6272
NPAD = NS * CHUNK
TAIL = NPAD - N
REAL_LAST = CHUNK - TAIL
NV = CHUNK // L
NVH = NV // 2
HBASE = NVH * L
NBINS = 256
INT_MIN = jnp.int32(-2**31)

_mesh = None


def _get_mesh():
    global _mesh
    if _mesh is None:
        _mesh = plsc.VectorSubcoreMesh(
            core_axis_name="c", subcore_axis_name="s",
            num_cores=NC, num_subcores=NS)
    return _mesh


def _body(x_hbm, out_hbm, keys_sp, perm_a, perm_b, totals_sp,
          fbuf, kbuf, pbuf, posbuf, hist0, hist1, hsum, offsa, offsb, ttbuf):
    c = lax.axis_index("c")
    t = lax.axis_index("s")
    lane = lax.iota(jnp.int32, L)
    zeros16 = jnp.zeros((L,), jnp.int32)
    my_lo = t * CHUNK

    def digits(k, shift):
        return lax.shift_right_logical(k, shift) & 255

    def radix_pass(shift, perm_in, perm_out, first):
        if not first:
            pltpu.sync_copy(perm_in.at[pl.ds(my_lo, CHUNK)], pbuf)
            pltpu.sync_copy(keys_sp.at[pbuf], kbuf)

        for j in range(NBINS // L):
            hist0[pl.ds(L * j, L)] = zeros16
            hist1[pl.ds(L * j, L)] = zeros16

        def hist_body(i, _):
            d0 = digits(kbuf[pl.ds(L * i, L)], shift)
            d1 = digits(kbuf[pl.ds(HBASE + L * i, L)], shift)
            c0, l0 = plsc.scan_count(d0)
            c1, l1 = plsc.scan_count(d1)
            plsc.addupdate_scatter(hist0, [d0], c0, mask=l0)
            plsc.addupdate_scatter(hist1, [d1], c1, mask=l1)
            return 0
        lax.fori_loop(0, NVH, hist_body, 0)

        for j in range(NBINS // L):
            sl = pl.ds(L * j, L)
            hsum[sl] = hist0[sl] + hist1[sl]
        pltpu.sync_copy(hsum, totals_sp.at[t])
        plsc.subcore_barrier()
        pltpu.sync_copy(totals_sp, ttbuf)

        carry = jnp.int32(0)
        for j in range(NBINS // L):
            sl = pl.ds(L * j, L)

            def tot_body(tp, acc):
                return acc + ttbuf[tp, sl]
            tot = lax.fori_loop(0, NS, tot_body, zeros16)
            below = lax.fori_loop(0, t, tot_body, zeros16)
            excl = plsc.cumsum(tot) - tot
            offsa[sl] = excl + below + carry
            offsb[sl] = excl + below + carry + hist0[sl]
            carry = carry + jnp.sum(tot)

        def pos_body(i, _):
            d0 = digits(kbuf[pl.ds(L * i, L)], shift)
            d1 = digits(kbuf[pl.ds(HBASE + L * i, L)], shift)
            c0, l0 = plsc.scan_count(d0)
            c1, l1 = plsc.scan_count(d1)
            o0 = plsc.load_gather(offsa, [d0])
            o1 = plsc.load_gather(offsb, [d1])
            posbuf[pl.ds(L * i, L)] = o0 + c0 - 1
            posbuf[pl.ds(HBASE + L * i, L)] = o1 + c1 - 1
            plsc.addupdate_scatter(offsa, [d0], c0, mask=l0)
            plsc.addupdate_scatter(offsb, [d1], c1, mask=l1)
            return 0
        lax.fori_loop(0, NVH, pos_body, 0)

        pltpu.sync_copy(pbuf, perm_out.at[posbuf])
        plsc.subcore_barrier()

    def row_body(r, _):
        row = c * ROWS_PER_SC + r

        @pl.when(t < NS - 1)
        def _():
            pltpu.sync_copy(x_hbm.at[row, pl.ds(my_lo, CHUNK)], fbuf)

        @pl.when(t == NS - 1)
        def _():
            pltpu.sync_copy(x_hbm.at[row, pl.ds((NS - 1) * CHUNK, REAL_LAST)],
                            fbuf.at[pl.ds(0, REAL_LAST)])

        def xform_body(i, _):
            b = plsc.bitcast(fbuf[pl.ds(L * i, L)], jnp.int32)
            key = b ^ (lax.shift_right_arithmetic(b, 31) | INT_MIN)
            kbuf[pl.ds(L * i, L)] = key
            pbuf[pl.ds(L * i, L)] = my_lo + L * i + lane
            return 0
        lax.fori_loop(0, NV, xform_body, 0)

        @pl.when(t == NS - 1)
        def _():
            for j in range(TAIL // L):
                kbuf[pl.ds(REAL_LAST + L * j, L)] = zeros16 - 1

        pltpu.sync_copy(kbuf, keys_sp.at[pl.ds(my_lo, CHUNK)])

        radix_pass(0, perm_a, perm_b, first=True)
        radix_pass(8, perm_b, perm_a, first=False)
        radix_pass(16, perm_a, perm_b, first=False)
        radix_pass(24, perm_b, perm_a, first=False)

        @pl.when(t < NS - 1)
        def _():
            pltpu.sync_copy(perm_a.at[pl.ds(my_lo, CHUNK)],
                            out_hbm.at[row, pl.ds(my_lo, CHUNK)])

        @pl.when(t == NS - 1)
        def _():
            pltpu.sync_copy(perm_a.at[pl.ds((NS - 1) * CHUNK, REAL_LAST)],
                            out_hbm.at[row, pl.ds((NS - 1) * CHUNK, REAL_LAST)])
        return 0

    lax.fori_loop(0, ROWS_PER_SC, row_body, 0)


@jax.jit
def kernel(x):
    run = functools.partial(
        pl.kernel,
        out_type=jax.ShapeDtypeStruct((ROWS, N), jnp.int32),
        mesh=_get_mesh(),
        scratch_types=[
            pltpu.VMEM_SHARED((NPAD,), jnp.int32),
            pltpu.VMEM_SHARED((NPAD,), jnp.int32),
            pltpu.VMEM_SHARED((NPAD,), jnp.int32),
            pltpu.VMEM_SHARED((NS, NBINS), jnp.int32),
            pltpu.VMEM((CHUNK,), jnp.float32),
            pltpu.VMEM((CHUNK,), jnp.int32),
            pltpu.VMEM((CHUNK,), jnp.int32),
            pltpu.VMEM((CHUNK,), jnp.int32),
            pltpu.VMEM((NBINS,), jnp.int32),
            pltpu.VMEM((NBINS,), jnp.int32),
            pltpu.VMEM((NBINS,), jnp.int32),
            pltpu.VMEM((NBINS,), jnp.int32),
            pltpu.VMEM((NBINS,), jnp.int32),
            pltpu.VMEM((NS, NBINS), jnp.int32),
        ],
        compiler_params=pltpu.CompilerParams(
            needs_layout_passes=False, use_tc_tiling_on_sc=False),
    )(_body)
    return run(x)
